# Initial kernel scaffold; baseline (speedup 1.0000x reference)
import jax
import jax.numpy as jnp
from jax.experimental import pallas as pl


def kernel(xyz, features):
    raise NotImplementedError("write your pallas kernel here")


# speedup vs baseline: 15.7464x; 15.7464x over previous
import functools

import jax
import jax.numpy as jnp
from jax import lax
from jax.experimental import pallas as pl
from jax.experimental.pallas import tpu as pltpu

B = 8
K = 4096
C = 256
N = 512


def _fps_body(x_ref, y_ref, z_ref, inds_ref, nx_ref, ny_ref, nz_ref):
    x = x_ref[...]
    y = y_ref[...]
    z = z_ref[...]
    iota = lax.broadcasted_iota(jnp.int32, (B, K), 1)

    lx = x[:, 0:1]
    ly = y[:, 0:1]
    lz = z[:, 0:1]
    inds_ref[0:1, :] = jnp.zeros((1, B), jnp.int32)
    nx_ref[0:1, :] = lx.T
    ny_ref[0:1, :] = ly.T
    nz_ref[0:1, :] = lz.T

    dists0 = jnp.full((B, K), 1e10, dtype=jnp.float32)

    def body(i, carry):
        dists, lx, ly, lz = carry
        dx = x - lx
        dy = y - ly
        dz = z - lz
        d = dx * dx + dy * dy + dz * dz
        dists = jnp.minimum(dists, d)
        m = jnp.max(dists, axis=1, keepdims=True)
        idx = jnp.min(jnp.where(dists == m, iota, K), axis=1, keepdims=True)
        sel = iota == idx
        lx = jnp.sum(jnp.where(sel, x, 0.0), axis=1, keepdims=True)
        ly = jnp.sum(jnp.where(sel, y, 0.0), axis=1, keepdims=True)
        lz = jnp.sum(jnp.where(sel, z, 0.0), axis=1, keepdims=True)
        inds_ref[pl.ds(i, 1), :] = idx.T
        nx_ref[pl.ds(i, 1), :] = lx.T
        ny_ref[pl.ds(i, 1), :] = ly.T
        nz_ref[pl.ds(i, 1), :] = lz.T
        return dists, lx, ly, lz

    lax.fori_loop(1, N, body, (dists0, lx, ly, lz))


@jax.jit
def _fps(x, y, z):
    out_shapes = (
        jax.ShapeDtypeStruct((N, B), jnp.int32),
        jax.ShapeDtypeStruct((N, B), jnp.float32),
        jax.ShapeDtypeStruct((N, B), jnp.float32),
        jax.ShapeDtypeStruct((N, B), jnp.float32),
    )
    return pl.pallas_call(
        _fps_body,
        out_shape=out_shapes,
    )(x, y, z)


@jax.jit
def kernel(xyz, features):
    x = xyz[:, :, 0]
    y = xyz[:, :, 1]
    z = xyz[:, :, 2]
    inds_t, nx, ny, nz = _fps(x, y, z)
    sample_inds = inds_t.T
    new_xyz = jnp.stack([nx.T, ny.T, nz.T], axis=-1)
    new_features = jnp.take_along_axis(features, sample_inds[:, None, :], axis=2)
    return new_xyz, new_features, sample_inds
